# Initial kernel scaffold; baseline (speedup 1.0000x reference)
import jax
import jax.numpy as jnp
from jax.experimental import pallas as pl


def kernel(rec_na, rec_ea, rec_idx, params):
    raise NotImplementedError("write your pallas kernel here")


# speedup vs baseline: 13.3208x; 13.3208x over previous
import functools
import math

import jax
import jax.numpy as jnp
import numpy as np
from jax import lax
from jax.experimental import pallas as pl
from jax.experimental.pallas import tpu as pltpu
from jax.experimental.pallas import tpu_sc as plsc

D = 128
H = 4
DH = D // H

NC = 2
NS = 16
NW = NC * NS

WROW = 144


def _gelu(x):
    return 0.5 * x * (1.0 + lax.erf(x * (1.0 / np.sqrt(2.0))))


def _make_gather(n_rows, e, ch):
    ew = e // NW
    nchunk = ew // ch
    mesh = plsc.VectorSubcoreMesh(core_axis_name="c", subcore_axis_name="s")

    @functools.partial(
        pl.kernel,
        out_type=jax.ShapeDtypeStruct((e, D), jnp.float32),
        mesh=mesh,
        scratch_types=[
            pltpu.VMEM((nchunk, ch), jnp.int32),
            pltpu.VMEM((ch, D), jnp.float32),
            pltpu.VMEM((ch, D), jnp.float32),
            pltpu.SemaphoreType.DMA,
            pltpu.SemaphoreType.DMA,
        ],
    )
    def gather_k(table_hbm, idx_hbm, out_hbm, idx_v, buf0, buf1, sem0, sem1):
        sid = lax.axis_index("s")
        wid = sid * NC + lax.axis_index("c")
        pltpu.sync_copy(idx_hbm.at[wid], idx_v)
        base = wid * ew

        cp0 = pltpu.async_copy(table_hbm.at[idx_v.at[0]], buf0, sem0)

        def body(c, _):
            buf_a, buf_b = buf0, buf1
            sem_a, sem_b = sem0, sem1
            even = lax.rem(c, 2) == 0

            def step(ba, sa, bb, sb):
                pltpu.make_async_copy(table_hbm.at[idx_v.at[c]], ba, sa).wait()

                cc = jnp.minimum(c + 1, nchunk - 1)

                @pl.when(c + 1 < nchunk)
                def _():
                    pltpu.async_copy(table_hbm.at[idx_v.at[cc]], bb, sb)

                pltpu.sync_copy(ba, out_hbm.at[pl.ds(base + c * ch, ch)])

            @pl.when(even)
            def _():
                step(buf_a, sem_a, buf_b, sem_b)

            @pl.when(jnp.logical_not(even))
            def _():
                step(buf_b, sem_b, buf_a, sem_a)

            return 0

        lax.fori_loop(0, nchunk, body, 0)

    return gather_k


def _make_scatter(e, ch, npad):
    ew = e // NW
    nchunk = ew // ch
    rows_per_tile = npad // NS
    mesh = plsc.VectorSubcoreMesh(core_axis_name="c", subcore_axis_name="s")

    stage_offs = []
    off = 0
    while off < rows_per_tile:
        sz = min(ch, rows_per_tile - off)
        stage_offs.append((off, sz))
        off += sz

    @functools.partial(
        pl.kernel,
        out_type=(jax.ShapeDtypeStruct((NC * npad, D), jnp.float32),
                  jax.ShapeDtypeStruct((NC * npad, 16), jnp.float32)),
        mesh=mesh,
        scratch_types=[
            pltpu.VMEM((ch,), jnp.int32),
            pltpu.VMEM((ch, D), jnp.float32),
            pltpu.VMEM((ch, 16), jnp.float32),
            pltpu.VMEM_SHARED((npad, D), jnp.float32),
            pltpu.VMEM_SHARED((npad, 16), jnp.float32),
        ],
    )
    def scatter_k(wv_hbm, ex_hbm, idx_hbm, z128_hbm, z16_hbm,
                  out_hbm, dout_hbm, idxc, wbuf, exbuf, acc, dacc):
        cid = lax.axis_index("c")
        sid = lax.axis_index("s")
        wid = sid * NC + cid
        r0 = sid * rows_per_tile
        pltpu.sync_copy(z128_hbm.at[pl.ds(0, ch)], wbuf)
        pltpu.sync_copy(z16_hbm.at[pl.ds(0, ch)], exbuf)
        for q, sz in stage_offs:
            pltpu.sync_copy(wbuf.at[pl.ds(0, sz)],
                            acc.at[pl.ds(r0 + q, sz)])
            pltpu.sync_copy(exbuf.at[pl.ds(0, sz)],
                            dacc.at[pl.ds(r0 + q, sz)])
        plsc.subcore_barrier()
        base = wid * ew

        def body(c, _):
            pltpu.sync_copy(idx_hbm.at[wid * nchunk + c], idxc)
            pltpu.sync_copy(wv_hbm.at[pl.ds(base + c * ch, ch)], wbuf)
            pltpu.sync_copy(ex_hbm.at[pl.ds(base + c * ch, ch)], exbuf)
            pltpu.sync_copy(wbuf, acc.at[idxc], add=True)
            pltpu.sync_copy(exbuf, dacc.at[idxc], add=True)
            return 0

        lax.fori_loop(0, nchunk, body, 0)
        plsc.subcore_barrier()
        for q, sz in stage_offs:
            pltpu.sync_copy(acc.at[pl.ds(r0 + q, sz)], wbuf.at[pl.ds(0, sz)])
            pltpu.sync_copy(wbuf.at[pl.ds(0, sz)],
                            out_hbm.at[pl.ds(cid * npad + r0 + q, sz)])
            pltpu.sync_copy(dacc.at[pl.ds(r0 + q, sz)],
                            exbuf.at[pl.ds(0, sz)])
            pltpu.sync_copy(exbuf.at[pl.ds(0, sz)],
                            dout_hbm.at[pl.ds(cid * npad + r0 + q, sz)])

    return scatter_k


def _edge1_body(sg_ref, dg_ref, ea_ref, b1w_ref, b1b_ref, b2w_ref, b2b_ref,
                b3w_ref, b3b_ref, v1w_ref, v1b_ref, v2w_ref, v2b_ref,
                v3w_ref, v3b_ref, v_ref, l_ref, c_ref):
    i = pl.program_id(0)
    sg = sg_ref[...]
    dg = dg_ref[...]
    ea = ea_ref[...]
    b1w = b1w_ref[...]
    hb = sg @ b1w[0:D] + ea @ b1w[D:2 * D] + dg @ b1w[2 * D:3 * D]
    hb = jnp.maximum(hb + b1b_ref[...], 0.0)
    hb = jnp.maximum(hb @ b2w_ref[...] + b2b_ref[...], 0.0)
    logits = (hb @ b3w_ref[...] + b3b_ref[...]) * (1.0 / np.sqrt(DH))
    l_ref[...] = logits
    v1w = v1w_ref[...]
    v = _gelu(sg @ v1w[0:D] + ea @ v1w[D:2 * D] + v1b_ref[...])
    v = _gelu(v @ v2w_ref[...] + v2b_ref[...])
    v_ref[...] = v @ v3w_ref[...] + v3b_ref[...]

    @pl.when(i == 0)
    def _():
        c_ref[...] = jnp.full((8, D), -jnp.inf, jnp.float32)

    c_ref[...] = jnp.maximum(c_ref[...], jnp.max(logits))


def _expmul_body(v_ref, l_ref, c_ref, r128_ref, m16_ref, wv_ref, ex_ref):
    ex8 = jnp.exp(l_ref[...] - c_ref[0:1, 0:8])
    ex128 = ex8 @ r128_ref[...]
    wv_ref[...] = ex128 * v_ref[...]
    ex_ref[...] = ex8 @ m16_ref[...]


def _node_body(p_ref, dp_ref, na_ref, r16128_ref, wo_ref, bn1g_ref, bn1b_ref,
               d1w_ref, d1b_ref, d2w_ref, d2b_ref, bn2g_ref, bn2b_ref,
               x_ref, n_rows):
    numer = (p_ref[0] + p_ref[1])[0:n_rows, :]
    den16 = jnp.sum(dp_ref[...], axis=0)[0:n_rows, :]
    den128 = den16 @ r16128_ref[...]
    den_safe = den128 + jnp.where(den128 == 0.0, 1.0, 0.0)
    agg = numer / den_safe
    dh = agg @ wo_ref[...]
    xr = na_ref[...] + dh
    m = jnp.mean(xr, axis=0, keepdims=True)
    v = jnp.mean((xr - m) ** 2, axis=0, keepdims=True)
    x = bn1g_ref[...] * (xr - m) * lax.rsqrt(v + 1e-5) + bn1b_ref[...]
    h = jnp.maximum(x @ d1w_ref[...] + d1b_ref[...], 0.0)
    dh2 = h @ d2w_ref[...] + d2b_ref[...]
    xr2 = x + dh2
    m2 = jnp.mean(xr2, axis=0, keepdims=True)
    v2 = jnp.mean((xr2 - m2) ** 2, axis=0, keepdims=True)
    x_ref[...] = (bn2g_ref[...] * (xr2 - m2) * lax.rsqrt(v2 + 1e-5)
                  + bn2b_ref[...])


def _edge2_body(sg_ref, ea_ref, xd_ref, w11_ref, b11_ref, w12_ref, b12_ref,
                w13_ref, b13_ref, t_ref, s_ref):
    i = pl.program_id(0)
    sg = sg_ref[...]
    ea = ea_ref[...]
    xd = xd_ref[...]
    w11 = w11_ref[...]
    m = _gelu(sg @ w11[0:D] + ea @ w11[D:2 * D] + xd @ w11[2 * D:3 * D]
              + b11_ref[...])
    m = _gelu(m @ w12_ref[...] + b12_ref[...])
    m = m @ w13_ref[...] + b13_ref[...]
    t = ea + m
    t_ref[...] = t

    @pl.when(i == 0)
    def _():
        s_ref[...] = jnp.zeros((8, D), jnp.float32)

    s_ref[0:1, :] += jnp.sum(t, axis=0, keepdims=True)
    s_ref[1:2, :] += jnp.sum(t * t, axis=0, keepdims=True)


def _norm_body(t_ref, s_ref, g_ref, b_ref, o_ref, e_total):
    mean = s_ref[0:1, :] * (1.0 / e_total)
    var = s_ref[1:2, :] * (1.0 / e_total) - mean * mean
    o_ref[...] = (g_ref[...] * (t_ref[...] - mean) * lax.rsqrt(var + 1e-5)
                  + b_ref[...])


def _full(shape):
    return pl.BlockSpec(shape, lambda i: (0,) * len(shape))


def kernel(rec_na, rec_ea, rec_idx, params):
    p = params
    n = rec_na.shape[0]
    e = rec_ea.shape[0]
    dst = rec_idx[0]
    src = rec_idx[1]

    ch = 80
    npad = ((n + NS * 8 - 1) // (NS * 8)) * NS * 8
    blk = 2000
    grid = e // blk

    idx3_dst = dst.reshape(NW, (e // NW) // ch, ch)
    idx3_src = src.reshape(NW, (e // NW) // ch, ch)

    gather = _make_gather(n, e, ch)
    src_g = gather(rec_na, idx3_src)
    dst_g = gather(rec_na, idx3_dst)

    b3w = jnp.pad(p['att_b3_w'], ((0, 0), (0, 8 - H)))
    b3b = jnp.pad(p['att_b3_b'], (0, 8 - H)).reshape(1, 8)

    row = _full((1, D))
    wblk = pl.BlockSpec((blk, D), lambda i: (i, 0))

    v_arr, l_arr, c_arr = pl.pallas_call(
        _edge1_body,
        grid=(grid,),
        in_specs=[wblk, wblk, wblk,
                  _full((3 * D, D)), row, _full((D, D)), row,
                  _full((D, 8)), _full((1, 8)),
                  _full((2 * D, D)), row, _full((D, D)), row,
                  _full((D, D)), row],
        out_specs=[wblk, pl.BlockSpec((blk, 8), lambda i: (i, 0)),
                   _full((8, D))],
        out_shape=[jax.ShapeDtypeStruct((e, D), jnp.float32),
                   jax.ShapeDtypeStruct((e, 8), jnp.float32),
                   jax.ShapeDtypeStruct((8, D), jnp.float32)],
    )(src_g, dst_g, rec_ea,
      p['att_b1_w'], p['att_b1_b'].reshape(1, D),
      p['att_b2_w'], p['att_b2_b'].reshape(1, D),
      b3w, b3b,
      p['att_wv1_w'], p['att_wv1_b'].reshape(1, D),
      p['att_wv2_w'], p['att_wv2_b'].reshape(1, D),
      p['att_wv3_w'], p['att_wv3_b'].reshape(1, D))

    r128 = np.zeros((8, D), np.float32)
    for h in range(H):
        r128[h, h * DH:(h + 1) * DH] = 1.0
    m16 = np.zeros((8, 16), np.float32)
    for h in range(H):
        m16[h, h] = 1.0

    wv_arr, ex_arr = pl.pallas_call(
        _expmul_body,
        grid=(grid,),
        in_specs=[wblk, pl.BlockSpec((blk, 8), lambda i: (i, 0)),
                  _full((8, D)), _full((8, D)), _full((8, 16))],
        out_specs=[wblk, pl.BlockSpec((blk, 16), lambda i: (i, 0))],
        out_shape=[jax.ShapeDtypeStruct((e, D), jnp.float32),
                   jax.ShapeDtypeStruct((e, 16), jnp.float32)],
    )(v_arr, l_arr, c_arr, jnp.asarray(r128), jnp.asarray(m16))

    r16128 = np.zeros((16, D), np.float32)
    for h in range(H):
        r16128[h, h * DH:(h + 1) * DH] = 1.0

    num = jax.ops.segment_sum(wv_arr, dst, num_segments=npad)
    den = jax.ops.segment_sum(ex_arr, dst, num_segments=npad)
    partials = jnp.stack([num, jnp.zeros_like(num)])
    dpart = jnp.stack([den, jnp.zeros_like(den)])

    x = pl.pallas_call(
        functools.partial(_node_body, n_rows=n),
        grid=(1,),
        in_specs=[_full((NC, npad, D)), _full((NC, npad, 16)), _full((n, D)),
                  _full((16, D)),
                  _full((D, D)), row, row,
                  _full((D, 4 * D)), _full((1, 4 * D)),
                  _full((4 * D, D)), row, row, row],
        out_specs=_full((n, D)),
        out_shape=jax.ShapeDtypeStruct((n, D), jnp.float32),
    )(partials, dpart, rec_na, jnp.asarray(r16128),
      p['att_wo_w'], p['bn1_g'].reshape(1, D), p['bn1_b'].reshape(1, D),
      p['dense1_w'], p['dense1_b'].reshape(1, 4 * D),
      p['dense2_w'], p['dense2_b'].reshape(1, D),
      p['bn2_g'].reshape(1, D), p['bn2_b'].reshape(1, D))

    xd = gather(x, idx3_dst)

    t_arr, s_arr = pl.pallas_call(
        _edge2_body,
        grid=(grid,),
        in_specs=[wblk, wblk, wblk,
                  _full((3 * D, D)), row, _full((D, D)), row,
                  _full((D, D)), row],
        out_specs=[wblk, _full((8, D))],
        out_shape=[jax.ShapeDtypeStruct((e, D), jnp.float32),
                   jax.ShapeDtypeStruct((8, D), jnp.float32)],
    )(src_g, rec_ea, xd,
      p['e_w11_w'], p['e_w11_b'].reshape(1, D),
      p['e_w12_w'], p['e_w12_b'].reshape(1, D),
      p['e_w13_w'], p['e_w13_b'].reshape(1, D))

    ea_out = pl.pallas_call(
        functools.partial(_norm_body, e_total=float(e)),
        grid=(grid,),
        in_specs=[wblk, _full((8, D)), row, row],
        out_specs=wblk,
        out_shape=jax.ShapeDtypeStruct((e, D), jnp.float32),
    )(t_arr, s_arr, p['e_bn_g'].reshape(1, D), p['e_bn_b'].reshape(1, D))

    return x, ea_out
